# Initial kernel scaffold; baseline (speedup 1.0000x reference)
import jax
import jax.numpy as jnp
from jax.experimental import pallas as pl


def kernel(x, edge_index, W, b):
    raise NotImplementedError("write your pallas kernel here")


# speedup vs baseline: 10.4065x; 10.4065x over previous
import functools

import jax
import jax.numpy as jnp
from jax import lax
from jax.experimental import pallas as pl
from jax.experimental.pallas import tpu as pltpu
from jax.experimental.pallas import tpu_sc as plsc

K_HOPS = 10
ALPHA = 0.05
NCORES = 2
NSUB = 16
CH = 128
IC = 8
RB = 80
RPT = 8


def _f32v(val):
    return jnp.full((16,), val, dtype=jnp.float32)


def _ssgc_sc(x0, x1, row2d, col2d, *, n, d, cpb):
    fpc = d // NCORES
    c1 = (1.0 - ALPHA) / K_HOPS
    nch_real = n // RB
    npad = -(-(n + 1) // RB) * RB
    nzch = npad // RB
    ndeg = NSUB * RB * RPT

    mesh = plsc.VectorSubcoreMesh(
        core_axis_name="c", subcore_axis_name="s", num_cores=NCORES,
        num_subcores=NSUB)

    @functools.partial(
        pl.kernel,
        out_type=[jax.ShapeDtypeStruct((n, fpc), jnp.float32),
                  jax.ShapeDtypeStruct((n, fpc), jnp.float32)],
        mesh=mesh,
        compiler_params=pltpu.CompilerParams(use_tc_tiling_on_sc=False),
        scratch_types=[
            pltpu.VMEM_SHARED((npad, fpc), jnp.float32),
            pltpu.VMEM_SHARED((npad, fpc), jnp.float32),
            pltpu.VMEM_SHARED((ndeg,), jnp.float32),
            pltpu.VMEM((IC, CH), jnp.int32),
            pltpu.VMEM((IC, CH), jnp.int32),
            pltpu.VMEM((CH, fpc), jnp.float32),
            pltpu.VMEM((RB, fpc), jnp.float32),
            pltpu.VMEM((RB, fpc), jnp.float32),
            pltpu.VMEM((RB, fpc), jnp.float32),
            pltpu.VMEM((RB, fpc), jnp.float32),
            pltpu.VMEM((CH,), jnp.float32),
            pltpu.VMEM((RB * RPT,), jnp.float32),
            pltpu.VMEM((RB * RPT + 16,), jnp.float32),
            pltpu.VMEM((RB * RPT + 16,), jnp.float32),
            pltpu.VMEM((RB * RPT + 16,), jnp.float32),
            pltpu.VMEM((RB * RPT,), jnp.float32),
        ],
    )
    def k(x0_hbm, x1_hbm, row_hbm, col_hbm, h0_hbm, h1_hbm,
          z_sh, u_sh, deg_sh,
          rowc, colc, gbuf, ubuf, zbuf, szbuf, zerob, onesb,
          degb, dinvb, d2b, dsqb, zdegb):
        cid = lax.axis_index("c")
        sid = lax.axis_index("s")
        r0 = sid * (RB * RPT)
        g0 = sid * cpb
        nreal = jnp.minimum(RPT, jnp.maximum(0, nch_real - sid * RPT))
        nzero = jnp.minimum(RPT, jnp.maximum(0, nzch - sid * RPT))

        def fill_const(i, carry):
            def fill_f(f, carry2):
                zerob[i, pl.ds(f * 16, 16)] = _f32v(0.0)
                return carry2
            return lax.fori_loop(0, fpc // 16, fill_f, carry, unroll=False)
        lax.fori_loop(0, RB, fill_const, 0, unroll=False)

        def fill_ones(i, carry):
            onesb[pl.ds(i * 16, 16)] = _f32v(1.0)
            return carry
        lax.fori_loop(0, CH // 16, fill_ones, 0, unroll=False)

        def fill_zdeg(i, carry):
            zdegb[pl.ds(i * 16, 16)] = _f32v(0.0)
            return carry
        lax.fori_loop(0, RB * RPT // 16, fill_zdeg, 0, unroll=False)

        pltpu.sync_copy(zdegb, deg_sh.at[pl.ds(r0, RB * RPT)])

        def zero_tables(c, carry):
            rr = r0 + c * RB
            pltpu.sync_copy(zerob, z_sh.at[pl.ds(rr, RB), :])
            pltpu.sync_copy(zerob, u_sh.at[pl.ds(rr, RB), :])
            return carry
        lax.fori_loop(0, nzero, zero_tables, 0, unroll=False)
        plsc.subcore_barrier()

        def deg_group(g, carry):
            pltpu.sync_copy(col_hbm.at[pl.ds(g0 + g * IC, IC), :], colc)
            def deg_chunk(j, carry2):
                pltpu.sync_copy(onesb, deg_sh.at[colc.at[j]], add=True)
                return carry2
            return lax.fori_loop(0, IC, deg_chunk, carry, unroll=False)
        lax.fori_loop(0, cpb // IC, deg_group, 0, unroll=False)
        plsc.subcore_barrier()

        pltpu.sync_copy(deg_sh.at[pl.ds(r0, RB * RPT)], degb)
        def rsqrt_vec(v, carry):
            dv = degb[pl.ds(v * 16, 16)] + _f32v(1.0)
            iv = lax.bitcast_convert_type(dv, jnp.int32)
            iv = jnp.int32(0x5F3759DF) - lax.shift_right_arithmetic(
                iv, jnp.int32(1))
            y = lax.bitcast_convert_type(iv, jnp.float32)
            half_d = dv * _f32v(0.5)
            y = y * (_f32v(1.5) - half_d * y * y)
            y = y * (_f32v(1.5) - half_d * y * y)
            y = y * (_f32v(1.5) - half_d * y * y)
            dinvb[pl.ds(v * 16, 16)] = y
            d2b[pl.ds(v * 16, 16)] = _f32v(1.0) / dv
            dsqb[pl.ds(v * 16, 16)] = dv * y * _f32v(c1)
            return carry
        lax.fori_loop(0, RB * RPT // 16, rsqrt_vec, 0, unroll=False)

        a_over_c = ALPHA / c1
        def z0_chunk(c, carry):
            rr = r0 + c * RB
            @pl.when(cid == 0)
            def _():
                pltpu.sync_copy(x0_hbm.at[pl.ds(rr, RB), :], zbuf)
            @pl.when(cid == 1)
            def _():
                pltpu.sync_copy(x1_hbm.at[pl.ds(rr, RB), :], zbuf)
            def z0_row(i, carry2):
                s = dinvb[pl.ds(c * RB + i, 16)][0]
                def z0_f(f, carry3):
                    sl = pl.ds(f * 16, 16)
                    zv = zbuf[i, sl] * s
                    zbuf[i, sl] = zv
                    szbuf[i, sl] = zv * a_over_c
                    return carry3
                return lax.fori_loop(0, fpc // 16, z0_f, carry2, unroll=False)
            lax.fori_loop(0, RB, z0_row, 0, unroll=False)
            pltpu.sync_copy(zbuf, z_sh.at[pl.ds(rr, RB), :])
            @pl.when(cid == 0)
            def _():
                pltpu.sync_copy(szbuf, h0_hbm.at[pl.ds(rr, RB), :])
            @pl.when(cid == 1)
            def _():
                pltpu.sync_copy(szbuf, h1_hbm.at[pl.ds(rr, RB), :])
            return carry
        lax.fori_loop(0, nreal, z0_chunk, 0, unroll=False)
        plsc.subcore_barrier()

        def hop(_, carry):
            def edge_group(g, carry2):
                pltpu.sync_copy(row_hbm.at[pl.ds(g0 + g * IC, IC), :], rowc)
                pltpu.sync_copy(col_hbm.at[pl.ds(g0 + g * IC, IC), :], colc)
                def edge_chunk(j, carry3):
                    pltpu.sync_copy(z_sh.at[rowc.at[j]], gbuf)
                    pltpu.sync_copy(gbuf, u_sh.at[colc.at[j]], add=True)
                    return carry3
                return lax.fori_loop(0, IC, edge_chunk, carry2, unroll=False)
            lax.fori_loop(0, cpb // IC, edge_group, 0, unroll=False)
            plsc.subcore_barrier()

            def scale_chunk(c, carry2):
                rr = r0 + c * RB
                pltpu.sync_copy(u_sh.at[pl.ds(rr, RB), :], ubuf)
                pltpu.sync_copy(z_sh.at[pl.ds(rr, RB), :], zbuf)
                @pl.when(cid == 0)
                def _():
                    pltpu.sync_copy(h0_hbm.at[pl.ds(rr, RB), :], szbuf)
                @pl.when(cid == 1)
                def _():
                    pltpu.sync_copy(h1_hbm.at[pl.ds(rr, RB), :], szbuf)
                def scale_row(i, carry3):
                    s = d2b[pl.ds(c * RB + i, 16)][0]
                    def scale_f(f, carry4):
                        sl = pl.ds(f * 16, 16)
                        zv = (ubuf[i, sl] + zbuf[i, sl]) * s
                        zbuf[i, sl] = zv
                        szbuf[i, sl] = szbuf[i, sl] + zv
                        return carry4
                    return lax.fori_loop(0, fpc // 16, scale_f, carry3,
                                         unroll=False)
                lax.fori_loop(0, RB, scale_row, 0, unroll=False)
                pltpu.sync_copy(zbuf, z_sh.at[pl.ds(rr, RB), :])
                pltpu.sync_copy(zerob, u_sh.at[pl.ds(rr, RB), :])
                @pl.when(cid == 0)
                def _():
                    pltpu.sync_copy(szbuf, h0_hbm.at[pl.ds(rr, RB), :])
                @pl.when(cid == 1)
                def _():
                    pltpu.sync_copy(szbuf, h1_hbm.at[pl.ds(rr, RB), :])
                return carry2
            lax.fori_loop(0, nreal, scale_chunk, 0, unroll=False)
            plsc.subcore_barrier()
            return carry
        lax.fori_loop(0, K_HOPS, hop, 0, unroll=False)

        def out_chunk(c, carry):
            rr = r0 + c * RB
            @pl.when(cid == 0)
            def _():
                pltpu.sync_copy(h0_hbm.at[pl.ds(rr, RB), :], szbuf)
            @pl.when(cid == 1)
            def _():
                pltpu.sync_copy(h1_hbm.at[pl.ds(rr, RB), :], szbuf)
            def out_row(i, carry2):
                s = dsqb[pl.ds(c * RB + i, 16)][0]
                def out_f(f, carry3):
                    sl = pl.ds(f * 16, 16)
                    szbuf[i, sl] = szbuf[i, sl] * s
                    return carry3
                return lax.fori_loop(0, fpc // 16, out_f, carry2,
                                     unroll=False)
            lax.fori_loop(0, RB, out_row, 0, unroll=False)
            @pl.when(cid == 0)
            def _():
                pltpu.sync_copy(szbuf, h0_hbm.at[pl.ds(rr, RB), :])
            @pl.when(cid == 1)
            def _():
                pltpu.sync_copy(szbuf, h1_hbm.at[pl.ds(rr, RB), :])
            return carry
        lax.fori_loop(0, nreal, out_chunk, 0, unroll=False)

    return k(x0, x1, row2d, col2d)


def _matmul_tc(h0, h1, w, b2, *, n, d, c):
    bn = 400
    hd = d // 2

    def mm(h0_ref, h1_ref, w_ref, b_ref, o_ref):
        wv = w_ref[...]
        o_ref[...] = (
            jnp.dot(h0_ref[...], wv[:hd], preferred_element_type=jnp.float32)
            + jnp.dot(h1_ref[...], wv[hd:], preferred_element_type=jnp.float32)
            + b_ref[...])

    return pl.pallas_call(
        mm,
        grid=(n // bn,),
        in_specs=[
            pl.BlockSpec((bn, hd), lambda i: (i, 0)),
            pl.BlockSpec((bn, hd), lambda i: (i, 0)),
            pl.BlockSpec((d, c), lambda i: (0, 0)),
            pl.BlockSpec((1, c), lambda i: (0, 0)),
        ],
        out_specs=pl.BlockSpec((bn, c), lambda i: (i, 0)),
        out_shape=jax.ShapeDtypeStruct((n, c), jnp.float32),
    )(h0, h1, w, b2)


def kernel(x, edge_index, W, b):
    n, d = x.shape
    e = edge_index.shape[1]
    c = W.shape[1]

    cpb = -(-e // (NSUB * CH))
    cpb = -(-cpb // IC) * IC
    epad = cpb * NSUB * CH
    pad = jnp.full((epad - e,), n, dtype=jnp.int32)
    row2d = jnp.concatenate([edge_index[0], pad]).reshape(cpb * NSUB, CH)
    col2d = jnp.concatenate([edge_index[1], pad]).reshape(cpb * NSUB, CH)

    hd = d // 2
    h0, h1 = _ssgc_sc(x[:, :hd], x[:, hd:], row2d, col2d, n=n, d=d, cpb=cpb)
    return _matmul_tc(h0, h1, W, b.reshape(1, c), n=n, d=d, c=c)


# speedup vs baseline: 14.7181x; 1.4143x over previous
import functools

import jax
import jax.numpy as jnp
from jax import lax
from jax.experimental import pallas as pl
from jax.experimental.pallas import tpu as pltpu
from jax.experimental.pallas import tpu_sc as plsc

K_HOPS = 10
ALPHA = 0.05
NCORES = 2
NSUB = 16
CH = 128
IC = 8
RB = 80
RPT = 8


def _f32v(val):
    return jnp.full((16,), val, dtype=jnp.float32)


def _ssgc_sc(x0, x1, row2d, col2d, *, n, d, cpb):
    fpc = d // NCORES
    c1 = (1.0 - ALPHA) / K_HOPS
    nch_real = n // RB
    npad = -(-(n + 1) // RB) * RB
    nzch = npad // RB
    ndeg = NSUB * RB * RPT

    mesh = plsc.VectorSubcoreMesh(
        core_axis_name="c", subcore_axis_name="s", num_cores=NCORES,
        num_subcores=NSUB)

    @functools.partial(
        pl.kernel,
        out_type=[jax.ShapeDtypeStruct((n, fpc), jnp.float32),
                  jax.ShapeDtypeStruct((n, fpc), jnp.float32)],
        mesh=mesh,
        compiler_params=pltpu.CompilerParams(use_tc_tiling_on_sc=False),
        scratch_types=[
            pltpu.VMEM_SHARED((npad, fpc), jnp.float32),
            pltpu.VMEM_SHARED((npad, fpc), jnp.float32),
            pltpu.VMEM_SHARED((ndeg,), jnp.float32),
            pltpu.VMEM((IC, CH), jnp.int32),
            pltpu.VMEM((IC, CH), jnp.int32),
            pltpu.VMEM((CH, fpc), jnp.float32),
            pltpu.VMEM((CH, fpc), jnp.float32),
            pltpu.VMEM((RB, fpc), jnp.float32),
            pltpu.VMEM((RB, fpc), jnp.float32),
            pltpu.VMEM((RB, fpc), jnp.float32),
            pltpu.VMEM((RB, fpc), jnp.float32),
            pltpu.VMEM((CH,), jnp.float32),
            pltpu.VMEM((RB * RPT,), jnp.float32),
            pltpu.VMEM((RB * RPT + 16,), jnp.float32),
            pltpu.VMEM((RB * RPT + 16,), jnp.float32),
            pltpu.VMEM((RB * RPT + 16,), jnp.float32),
            pltpu.VMEM((RB * RPT,), jnp.float32),
            pltpu.SemaphoreType.DMA,
            pltpu.SemaphoreType.DMA,
            pltpu.SemaphoreType.DMA,
            pltpu.SemaphoreType.DMA,
            pltpu.SemaphoreType.DMA,
        ],
    )
    def k(x0_hbm, x1_hbm, row_hbm, col_hbm, h0_hbm, h1_hbm,
          z_sh, u_sh, deg_sh,
          rowc, colc, gbufa, gbufb, ubuf, zbuf, szbuf, zerob, onesb,
          degb, dinvb, d2b, dsqb, zdegb, ssema, ssemb, sem0, sem1, sem2):
        cid = lax.axis_index("c")
        sid = lax.axis_index("s")
        r0 = sid * (RB * RPT)
        g0 = sid * cpb
        nreal = jnp.minimum(RPT, jnp.maximum(0, nch_real - sid * RPT))
        nzero = jnp.minimum(RPT, jnp.maximum(0, nzch - sid * RPT))

        def fill_const(i, carry):
            def fill_f(f, carry2):
                zerob[i, pl.ds(f * 16, 16)] = _f32v(0.0)
                return carry2
            return lax.fori_loop(0, fpc // 16, fill_f, carry, unroll=False)
        lax.fori_loop(0, RB, fill_const, 0, unroll=False)

        def fill_ones(i, carry):
            onesb[pl.ds(i * 16, 16)] = _f32v(1.0)
            return carry
        lax.fori_loop(0, CH // 16, fill_ones, 0, unroll=False)

        def fill_zdeg(i, carry):
            zdegb[pl.ds(i * 16, 16)] = _f32v(0.0)
            return carry
        lax.fori_loop(0, RB * RPT // 16, fill_zdeg, 0, unroll=False)

        pltpu.sync_copy(zdegb, deg_sh.at[pl.ds(r0, RB * RPT)])

        def zero_tables(c, carry):
            rr = r0 + c * RB
            pltpu.sync_copy(zerob, z_sh.at[pl.ds(rr, RB), :])
            pltpu.sync_copy(zerob, u_sh.at[pl.ds(rr, RB), :])
            return carry
        lax.fori_loop(0, nzero, zero_tables, 0, unroll=False)
        plsc.subcore_barrier()

        def deg_group(g, carry):
            pltpu.sync_copy(col_hbm.at[pl.ds(g0 + g * IC, IC), :], colc)
            def deg_chunk(j, carry2):
                pltpu.sync_copy(onesb, deg_sh.at[colc.at[j]], add=True)
                return carry2
            return lax.fori_loop(0, IC, deg_chunk, carry, unroll=False)
        lax.fori_loop(0, cpb // IC, deg_group, 0, unroll=False)
        plsc.subcore_barrier()

        pltpu.sync_copy(deg_sh.at[pl.ds(r0, RB * RPT)], degb)
        def rsqrt_vec(v, carry):
            dv = degb[pl.ds(v * 16, 16)] + _f32v(1.0)
            iv = lax.bitcast_convert_type(dv, jnp.int32)
            iv = jnp.int32(0x5F3759DF) - lax.shift_right_arithmetic(
                iv, jnp.int32(1))
            y = lax.bitcast_convert_type(iv, jnp.float32)
            half_d = dv * _f32v(0.5)
            y = y * (_f32v(1.5) - half_d * y * y)
            y = y * (_f32v(1.5) - half_d * y * y)
            y = y * (_f32v(1.5) - half_d * y * y)
            dinvb[pl.ds(v * 16, 16)] = y
            d2b[pl.ds(v * 16, 16)] = _f32v(1.0) / dv
            dsqb[pl.ds(v * 16, 16)] = dv * y * _f32v(c1)
            return carry
        lax.fori_loop(0, RB * RPT // 16, rsqrt_vec, 0, unroll=False)

        a_over_c = ALPHA / c1
        def z0_chunk(c, carry):
            rr = r0 + c * RB
            @pl.when(cid == 0)
            def _():
                pltpu.sync_copy(x0_hbm.at[pl.ds(rr, RB), :], zbuf)
            @pl.when(cid == 1)
            def _():
                pltpu.sync_copy(x1_hbm.at[pl.ds(rr, RB), :], zbuf)
            def z0_row(i, carry2):
                s = dinvb[pl.ds(c * RB + i, 16)][0]
                def z0_f(f, carry3):
                    sl = pl.ds(f * 16, 16)
                    zv = zbuf[i, sl] * s
                    zbuf[i, sl] = zv
                    szbuf[i, sl] = zv * a_over_c
                    return carry3
                return lax.fori_loop(0, fpc // 16, z0_f, carry2, unroll=False)
            lax.fori_loop(0, RB, z0_row, 0, unroll=False)
            pltpu.sync_copy(zbuf, z_sh.at[pl.ds(rr, RB), :])
            @pl.when(cid == 0)
            def _():
                pltpu.sync_copy(szbuf, h0_hbm.at[pl.ds(rr, RB), :])
            @pl.when(cid == 1)
            def _():
                pltpu.sync_copy(szbuf, h1_hbm.at[pl.ds(rr, RB), :])
            return carry
        lax.fori_loop(0, nreal, z0_chunk, 0, unroll=False)
        plsc.subcore_barrier()

        gbufs = (gbufa, gbufb)
        ssems = (ssema, ssemb)

        def hop(_, carry):
            def edge_group(g, carry2):
                pltpu.sync_copy(row_hbm.at[pl.ds(g0 + g * IC, IC), :], rowc)
                pltpu.sync_copy(col_hbm.at[pl.ds(g0 + g * IC, IC), :], colc)
                for j in range(IC):
                    b = j % 2
                    def _wait(b=b, j=j):
                        pltpu.make_async_copy(
                            gbufs[b], u_sh.at[colc.at[j]], ssems[b]).wait()
                    if j >= 2:
                        _wait()
                    else:
                        pl.when(g >= 1)(_wait)
                    pltpu.sync_copy(z_sh.at[rowc.at[j]], gbufs[b])
                    pltpu.async_copy(gbufs[b], u_sh.at[colc.at[j]],
                                     ssems[b], add=True)
                return carry2
            lax.fori_loop(0, cpb // IC, edge_group, 0, unroll=False)
            pltpu.make_async_copy(
                gbufs[0], u_sh.at[colc.at[IC - 2]], ssems[0]).wait()
            pltpu.make_async_copy(
                gbufs[1], u_sh.at[colc.at[IC - 1]], ssems[1]).wait()
            plsc.subcore_barrier()

            def scale_chunk(c, carry2):
                rr = r0 + c * RB
                d0 = pltpu.async_copy(u_sh.at[pl.ds(rr, RB), :], ubuf, sem0)
                d1 = pltpu.async_copy(z_sh.at[pl.ds(rr, RB), :], zbuf, sem1)
                @pl.when(cid == 0)
                def _():
                    pltpu.async_copy(h0_hbm.at[pl.ds(rr, RB), :], szbuf, sem2)
                @pl.when(cid == 1)
                def _():
                    pltpu.async_copy(h1_hbm.at[pl.ds(rr, RB), :], szbuf, sem2)
                d0.wait()
                d1.wait()
                pltpu.make_async_copy(
                    h0_hbm.at[pl.ds(rr, RB), :], szbuf, sem2).wait()
                def scale_row(i, carry3):
                    s = d2b[pl.ds(c * RB + i, 16)][0]
                    def scale_f(f, carry4):
                        sl = pl.ds(f * 16, 16)
                        zv = (ubuf[i, sl] + zbuf[i, sl]) * s
                        zbuf[i, sl] = zv
                        szbuf[i, sl] = szbuf[i, sl] + zv
                        return carry4
                    return lax.fori_loop(0, fpc // 16, scale_f, carry3,
                                         unroll=False)
                lax.fori_loop(0, RB, scale_row, 0, unroll=False)
                d3 = pltpu.async_copy(zbuf, z_sh.at[pl.ds(rr, RB), :], sem0)
                d4 = pltpu.async_copy(zerob, u_sh.at[pl.ds(rr, RB), :], sem1)
                @pl.when(cid == 0)
                def _():
                    pltpu.async_copy(szbuf, h0_hbm.at[pl.ds(rr, RB), :], sem2)
                @pl.when(cid == 1)
                def _():
                    pltpu.async_copy(szbuf, h1_hbm.at[pl.ds(rr, RB), :], sem2)
                d3.wait()
                d4.wait()
                pltpu.make_async_copy(
                    szbuf, h0_hbm.at[pl.ds(rr, RB), :], sem2).wait()
                return carry2
            lax.fori_loop(0, nreal, scale_chunk, 0, unroll=False)
            plsc.subcore_barrier()
            return carry
        lax.fori_loop(0, K_HOPS, hop, 0, unroll=False)

        def out_chunk(c, carry):
            rr = r0 + c * RB
            @pl.when(cid == 0)
            def _():
                pltpu.sync_copy(h0_hbm.at[pl.ds(rr, RB), :], szbuf)
            @pl.when(cid == 1)
            def _():
                pltpu.sync_copy(h1_hbm.at[pl.ds(rr, RB), :], szbuf)
            def out_row(i, carry2):
                s = dsqb[pl.ds(c * RB + i, 16)][0]
                def out_f(f, carry3):
                    sl = pl.ds(f * 16, 16)
                    szbuf[i, sl] = szbuf[i, sl] * s
                    return carry3
                return lax.fori_loop(0, fpc // 16, out_f, carry2,
                                     unroll=False)
            lax.fori_loop(0, RB, out_row, 0, unroll=False)
            @pl.when(cid == 0)
            def _():
                pltpu.sync_copy(szbuf, h0_hbm.at[pl.ds(rr, RB), :])
            @pl.when(cid == 1)
            def _():
                pltpu.sync_copy(szbuf, h1_hbm.at[pl.ds(rr, RB), :])
            return carry
        lax.fori_loop(0, nreal, out_chunk, 0, unroll=False)

    return k(x0, x1, row2d, col2d)


def _matmul_tc(h0, h1, w, b2, *, n, d, c):
    bn = 400
    hd = d // 2

    def mm(h0_ref, h1_ref, w_ref, b_ref, o_ref):
        wv = w_ref[...]
        o_ref[...] = (
            jnp.dot(h0_ref[...], wv[:hd], preferred_element_type=jnp.float32)
            + jnp.dot(h1_ref[...], wv[hd:], preferred_element_type=jnp.float32)
            + b_ref[...])

    return pl.pallas_call(
        mm,
        grid=(n // bn,),
        in_specs=[
            pl.BlockSpec((bn, hd), lambda i: (i, 0)),
            pl.BlockSpec((bn, hd), lambda i: (i, 0)),
            pl.BlockSpec((d, c), lambda i: (0, 0)),
            pl.BlockSpec((1, c), lambda i: (0, 0)),
        ],
        out_specs=pl.BlockSpec((bn, c), lambda i: (i, 0)),
        out_shape=jax.ShapeDtypeStruct((n, c), jnp.float32),
    )(h0, h1, w, b2)


def kernel(x, edge_index, W, b):
    n, d = x.shape
    e = edge_index.shape[1]
    c = W.shape[1]

    cpb = -(-e // (NSUB * CH))
    cpb = -(-cpb // IC) * IC
    epad = cpb * NSUB * CH
    pad = jnp.full((epad - e,), n, dtype=jnp.int32)
    row2d = jnp.concatenate([edge_index[0], pad]).reshape(cpb * NSUB, CH)
    col2d = jnp.concatenate([edge_index[1], pad]).reshape(cpb * NSUB, CH)

    hd = d // 2
    h0, h1 = _ssgc_sc(x[:, :hd], x[:, hd:], row2d, col2d, n=n, d=d, cpb=cpb)
    return _matmul_tc(h0, h1, W, b.reshape(1, c), n=n, d=d, c=c)
